# Initial kernel scaffold; baseline (speedup 1.0000x reference)
import jax
import jax.numpy as jnp
from jax.experimental import pallas as pl


def kernel(n_feat, e_feat, edge_index, W0, b0, We1, be1, We2, be2, bc, W1, b1, gamma, beta_bn, Wy, by, Wy2, by2):
    raise NotImplementedError("write your pallas kernel here")


# speedup vs baseline: 2.7129x; 2.7129x over previous
import functools

import jax
import jax.numpy as jnp
from jax import lax
from jax.experimental import pallas as pl
from jax.experimental.pallas import tpu as pltpu
from jax.experimental.pallas import tpu_sc as plsc

N = 10000
E = 160000
D_IN = 128
HID = 32
E_IN = 16
E_HID = 128
STEPS = 2
ALPHA = 0.1
BETA = 1.0 / STEPS

SC_CORES = 2
SC_SUBCORES = 16
SC_W = SC_CORES * SC_SUBCORES
EPW = E // SC_W
GCH = 128
NCHM = 39
TAIL = EPW - NCHM * GCH
NPAD = 10240
NPT = NPAD // SC_SUBCORES

TILE_E = 1000


def _lin0_body(nf, w0t, b0, out):
    out[...] = jnp.maximum(
        jnp.dot(nf[...], w0t[...], preferred_element_type=jnp.float32) + b0[...],
        0.0)


def _edge_body(ef, h, we1t, be1, we2t, be2, rrep, msg):
    t = jnp.maximum(
        jnp.dot(ef[...], we1t[...], preferred_element_type=jnp.float32) + be1[...],
        0.0)
    wm = jnp.dot(t, we2t[...], preferred_element_type=jnp.float32) + be2[...]
    hr = jnp.dot(h[...], rrep[...], preferred_element_type=jnp.float32)
    prod = hr * wm
    t1 = prod[:, 0:128]
    for k in range(1, 8):
        t1 = t1 + prod[:, k * 128:(k + 1) * 128]
    s = t1[:, 0:64] + t1[:, 64:128]
    msg[...] = s[:, 0:32] + s[:, 32:64]


def _node_body(agg_a, agg_b, out, h0, w1t, b1, bc, new):
    conv = agg_a[...] + agg_b[...] + out[...] + bc[...]
    temp = ALPHA * conv + (1.0 - ALPHA) * h0[...]
    lin = jnp.dot(temp, w1t[...], preferred_element_type=jnp.float32) + b1[...]
    new[...] = jnp.maximum(BETA * lin + (1.0 - BETA) * temp, 0.0)


def _bn_body(x, gamma, beta_bn, wyt, by, wy2t, by2, y1, y2):
    v = x[...]
    mu = jnp.mean(v, axis=0, keepdims=True)
    d = v - mu
    var = jnp.mean(d * d, axis=0, keepdims=True)
    yb = d * (gamma[...] * lax.rsqrt(var + 1e-5)) + beta_bn[...]
    y1[...] = jax.nn.sigmoid(
        jnp.dot(yb, wyt[...], preferred_element_type=jnp.float32) + by[...])
    y2[...] = jax.nn.sigmoid(
        jnp.dot(yb, wy2t[...], preferred_element_type=jnp.float32) + by2[...])


def _lin0(n_feat, w0t, b0):
    return pl.pallas_call(
        _lin0_body,
        out_shape=jax.ShapeDtypeStruct((N, HID), jnp.float32),
    )(n_feat, w0t, b0)


def _edge(e_feat, h_src, we1t, be1, we2t, be2, rrep):
    grid = (E // TILE_E,)
    fixed = lambda i: (0, 0)
    return pl.pallas_call(
        _edge_body,
        grid=grid,
        in_specs=[
            pl.BlockSpec((TILE_E, E_IN), lambda i: (i, 0)),
            pl.BlockSpec((TILE_E, HID), lambda i: (i, 0)),
            pl.BlockSpec((E_IN, E_HID), fixed),
            pl.BlockSpec((1, E_HID), fixed),
            pl.BlockSpec((E_HID, HID * HID), fixed),
            pl.BlockSpec((1, HID * HID), fixed),
            pl.BlockSpec((HID, HID * HID), fixed),
        ],
        out_specs=pl.BlockSpec((TILE_E, HID), lambda i: (i, 0)),
        out_shape=jax.ShapeDtypeStruct((E, HID), jnp.float32),
        compiler_params=pltpu.CompilerParams(
            dimension_semantics=("arbitrary",)),
    )(e_feat, h_src, we1t, be1, we2t, be2, rrep)


def _node(agg_a, agg_b, out, h0, w1t, b1, bc):
    return pl.pallas_call(
        _node_body,
        out_shape=jax.ShapeDtypeStruct((N, HID), jnp.float32),
    )(agg_a, agg_b, out, h0, w1t, b1, bc)


def _bn_heads(x, gamma, beta_bn, wyt, by, wy2t, by2):
    return pl.pallas_call(
        _bn_body,
        out_shape=(jax.ShapeDtypeStruct((N, 2), jnp.float32),
                   jax.ShapeDtypeStruct((N, 2), jnp.float32)),
    )(x, gamma, beta_bn, wyt, by, wy2t, by2)


def _sc_gather_body(table_hbm, idxm_hbm, idxt_hbm, out_hbm, idx_v, idxt_v,
                    rows_v, rowst_v, sem):
    c = lax.axis_index("c")
    s = lax.axis_index("s")
    wid = s * SC_CORES + c
    pltpu.sync_copy(idxm_hbm.at[wid], idx_v)
    pltpu.sync_copy(idxt_hbm.at[wid], idxt_v)

    def body(j, carry):
        pltpu.async_copy(table_hbm.at[idx_v.at[j]], rows_v, sem).wait()
        pltpu.sync_copy(rows_v, out_hbm.at[pl.ds(wid * EPW + j * GCH, GCH)])
        return carry

    lax.fori_loop(0, NCHM, body, 0)
    pltpu.async_copy(table_hbm.at[idxt_v], rowst_v, sem).wait()
    pltpu.sync_copy(rowst_v, out_hbm.at[pl.ds(wid * EPW + NCHM * GCH, TAIL)])


def _sc_scatter_body(msg_hbm, idxm_hbm, idxt_hbm, zer_hbm, out_hbm, idx_v,
                     idxt_v, rows_v, rowst_v, agg_sh, sem):
    c = lax.axis_index("c")
    s = lax.axis_index("s")
    wid = s * SC_CORES + c
    pltpu.sync_copy(zer_hbm, agg_sh.at[pl.ds(s * NPT, NPT)])
    pltpu.sync_copy(idxm_hbm.at[wid], idx_v)
    pltpu.sync_copy(idxt_hbm.at[wid], idxt_v)
    plsc.subcore_barrier()

    def body(j, carry):
        pltpu.async_copy(
            msg_hbm.at[pl.ds(wid * EPW + j * GCH, GCH)], rows_v, sem).wait()
        pltpu.sync_copy(rows_v, agg_sh.at[idx_v.at[j]], add=True)
        return carry

    lax.fori_loop(0, NCHM, body, 0)
    pltpu.async_copy(
        msg_hbm.at[pl.ds(wid * EPW + NCHM * GCH, TAIL)], rowst_v, sem).wait()
    pltpu.sync_copy(rowst_v, agg_sh.at[idxt_v], add=True)
    plsc.subcore_barrier()
    pltpu.sync_copy(agg_sh.at[pl.ds(s * NPT, NPT)],
                    out_hbm.at[pl.ds(c * NPAD + s * NPT, NPT)])


@functools.lru_cache(maxsize=None)
def _sc_kernels():
    mesh = plsc.VectorSubcoreMesh(core_axis_name="c", subcore_axis_name="s")
    params = pltpu.CompilerParams(use_tc_tiling_on_sc=False)
    gather = pl.kernel(
        _sc_gather_body,
        out_type=jax.ShapeDtypeStruct((E, HID), jnp.float32),
        mesh=mesh,
        scratch_types=[
            pltpu.VMEM((NCHM, GCH), jnp.int32),
            pltpu.VMEM((TAIL,), jnp.int32),
            pltpu.VMEM((GCH, HID), jnp.float32),
            pltpu.VMEM((TAIL, HID), jnp.float32),
            pltpu.SemaphoreType.DMA,
        ],
        compiler_params=params,
    )
    scatter = pl.kernel(
        _sc_scatter_body,
        out_type=jax.ShapeDtypeStruct((SC_CORES * NPAD, HID), jnp.float32),
        mesh=mesh,
        scratch_types=[
            pltpu.VMEM((NCHM, GCH), jnp.int32),
            pltpu.VMEM((TAIL,), jnp.int32),
            pltpu.VMEM((GCH, HID), jnp.float32),
            pltpu.VMEM((TAIL, HID), jnp.float32),
            pltpu.VMEM_SHARED((NPAD, HID), jnp.float32),
            pltpu.SemaphoreType.DMA,
        ],
        compiler_params=params,
    )
    return gather, scatter


def kernel(n_feat, e_feat, edge_index, W0, b0, We1, be1, We2, be2, bc, W1,
           b1, gamma, beta_bn, Wy, by, Wy2, by2):
    srcw = edge_index[0].reshape(SC_W, EPW)
    dstw = edge_index[1].reshape(SC_W, EPW)
    srcm = srcw[:, :NCHM * GCH].reshape(SC_W, NCHM, GCH)
    srct = srcw[:, NCHM * GCH:]
    dstm = dstw[:, :NCHM * GCH].reshape(SC_W, NCHM, GCH)
    dstt = dstw[:, NCHM * GCH:]

    w0t = W0.T
    we1t = We1.T
    we2t = We2.T
    w1t = W1.T
    wyt = Wy.T
    wy2t = Wy2.T
    b0r = b0.reshape(1, HID)
    be1r = be1.reshape(1, E_HID)
    be2r = be2.reshape(1, HID * HID)
    b1r = b1.reshape(1, HID)
    bcr = bc.reshape(1, HID)
    gr = gamma.reshape(1, HID)
    betar = beta_bn.reshape(1, HID)
    byr = by.reshape(1, 2)
    by2r = by2.reshape(1, 2)
    rrep = jnp.repeat(jnp.eye(HID, dtype=jnp.float32), HID, axis=1)
    zer = jnp.zeros((NPT, HID), jnp.float32)

    sc_gather, sc_scatter = _sc_kernels()
    out = _lin0(n_feat, w0t, b0r)
    h0 = out
    for _ in range(STEPS):
        h_src = sc_gather(out, srcm, srct)
        msg = _edge(e_feat, h_src, we1t, be1r, we2t, be2r, rrep)
        parts = sc_scatter(msg, dstm, dstt, zer).reshape(SC_CORES, NPAD, HID)
        out = _node(parts[0, :N], parts[1, :N], out, h0, w1t, b1r, bcr)
    return _bn_heads(out, gr, betar, wyt, byr, wy2t, by2r)


# speedup vs baseline: 2.9234x; 1.0776x over previous
import functools

import jax
import jax.numpy as jnp
from jax import lax
from jax.experimental import pallas as pl
from jax.experimental.pallas import tpu as pltpu
from jax.experimental.pallas import tpu_sc as plsc

N = 10000
E = 160000
D_IN = 128
HID = 32
E_IN = 16
E_HID = 128
STEPS = 2
ALPHA = 0.1
BETA = 1.0 / STEPS

SC_CORES = 2
SC_SUBCORES = 16
SC_W = SC_CORES * SC_SUBCORES
EPW = E // SC_W
GCH = 128
NCHM = 39
TAIL = EPW - NCHM * GCH
NPAD = 10240
NPT = NPAD // SC_SUBCORES

TILE_E = 2000


def _lin0_body(nf, w0t, b0, out):
    out[...] = jnp.maximum(
        jnp.dot(nf[...], w0t[...], preferred_element_type=jnp.float32) + b0[...],
        0.0)


def _edge_body(ef, h, we1t, be1, we2t, be2, rrep, msg):
    t = jnp.maximum(
        jnp.dot(ef[...], we1t[...], preferred_element_type=jnp.float32) + be1[...],
        0.0)
    wm = jnp.dot(t.astype(jnp.bfloat16), we2t[...],
                 preferred_element_type=jnp.float32) + be2[...]
    hr = jnp.dot(h[...].astype(jnp.bfloat16), rrep[...],
                 preferred_element_type=jnp.float32)
    prod = hr * wm
    t1 = prod[:, 0:128]
    for k in range(1, 8):
        t1 = t1 + prod[:, k * 128:(k + 1) * 128]
    s = t1[:, 0:64] + t1[:, 64:128]
    msg[...] = s[:, 0:32] + s[:, 32:64]


def _node_body(agg_a, agg_b, out, h0, w1t, b1, bc, new):
    conv = agg_a[...] + agg_b[...] + out[...] + bc[...]
    temp = ALPHA * conv + (1.0 - ALPHA) * h0[...]
    lin = jnp.dot(temp, w1t[...], preferred_element_type=jnp.float32) + b1[...]
    new[...] = jnp.maximum(BETA * lin + (1.0 - BETA) * temp, 0.0)


def _bn_body(x, gamma, beta_bn, wyt, by, wy2t, by2, y1, y2):
    v = x[...]
    mu = jnp.mean(v, axis=0, keepdims=True)
    d = v - mu
    var = jnp.mean(d * d, axis=0, keepdims=True)
    yb = d * (gamma[...] * lax.rsqrt(var + 1e-5)) + beta_bn[...]
    y1[...] = jax.nn.sigmoid(
        jnp.dot(yb, wyt[...], preferred_element_type=jnp.float32) + by[...])
    y2[...] = jax.nn.sigmoid(
        jnp.dot(yb, wy2t[...], preferred_element_type=jnp.float32) + by2[...])


def _lin0(n_feat, w0t, b0):
    return pl.pallas_call(
        _lin0_body,
        out_shape=jax.ShapeDtypeStruct((N, HID), jnp.float32),
    )(n_feat, w0t, b0)


def _edge(e_feat, h_src, we1t, be1, we2t, be2, rrep):
    grid = (E // TILE_E,)
    fixed = lambda i: (0, 0)
    return pl.pallas_call(
        _edge_body,
        grid=grid,
        in_specs=[
            pl.BlockSpec((TILE_E, E_IN), lambda i: (i, 0)),
            pl.BlockSpec((TILE_E, HID), lambda i: (i, 0)),
            pl.BlockSpec((E_IN, E_HID), fixed),
            pl.BlockSpec((1, E_HID), fixed),
            pl.BlockSpec((E_HID, HID * HID), fixed),
            pl.BlockSpec((1, HID * HID), fixed),
            pl.BlockSpec((HID, HID * HID), fixed),
        ],
        out_specs=pl.BlockSpec((TILE_E, HID), lambda i: (i, 0)),
        out_shape=jax.ShapeDtypeStruct((E, HID), jnp.float32),
        compiler_params=pltpu.CompilerParams(
            dimension_semantics=("arbitrary",)),
    )(e_feat, h_src, we1t, be1, we2t, be2, rrep)


def _node(agg_a, agg_b, out, h0, w1t, b1, bc):
    return pl.pallas_call(
        _node_body,
        out_shape=jax.ShapeDtypeStruct((N, HID), jnp.float32),
    )(agg_a, agg_b, out, h0, w1t, b1, bc)


def _bn_heads(x, gamma, beta_bn, wyt, by, wy2t, by2):
    return pl.pallas_call(
        _bn_body,
        out_shape=(jax.ShapeDtypeStruct((N, 2), jnp.float32),
                   jax.ShapeDtypeStruct((N, 2), jnp.float32)),
    )(x, gamma, beta_bn, wyt, by, wy2t, by2)


def _sc_gather_body(table_hbm, idxm_hbm, idxt_hbm, out_hbm, idx_v, idxt_v,
                    rows_v, rowst_v, sem):
    c = lax.axis_index("c")
    s = lax.axis_index("s")
    wid = s * SC_CORES + c
    pltpu.sync_copy(idxm_hbm.at[wid], idx_v)
    pltpu.sync_copy(idxt_hbm.at[wid], idxt_v)

    def body(j, carry):
        pltpu.async_copy(table_hbm.at[idx_v.at[j]], rows_v, sem).wait()
        pltpu.sync_copy(rows_v, out_hbm.at[pl.ds(wid * EPW + j * GCH, GCH)])
        return carry

    lax.fori_loop(0, NCHM, body, 0)
    pltpu.async_copy(table_hbm.at[idxt_v], rowst_v, sem).wait()
    pltpu.sync_copy(rowst_v, out_hbm.at[pl.ds(wid * EPW + NCHM * GCH, TAIL)])


def _sc_scatter_body(msg_hbm, idxm_hbm, idxt_hbm, zer_hbm, out_hbm, idx_v,
                     idxt_v, rows_v, rowst_v, agg_sh, sem):
    c = lax.axis_index("c")
    s = lax.axis_index("s")
    wid = s * SC_CORES + c
    pltpu.sync_copy(zer_hbm, agg_sh.at[pl.ds(s * NPT, NPT)])
    pltpu.sync_copy(idxm_hbm.at[wid], idx_v)
    pltpu.sync_copy(idxt_hbm.at[wid], idxt_v)
    plsc.subcore_barrier()

    def body(j, carry):
        pltpu.async_copy(
            msg_hbm.at[pl.ds(wid * EPW + j * GCH, GCH)], rows_v, sem).wait()
        pltpu.sync_copy(rows_v, agg_sh.at[idx_v.at[j]], add=True)
        return carry

    lax.fori_loop(0, NCHM, body, 0)
    pltpu.async_copy(
        msg_hbm.at[pl.ds(wid * EPW + NCHM * GCH, TAIL)], rowst_v, sem).wait()
    pltpu.sync_copy(rowst_v, agg_sh.at[idxt_v], add=True)
    plsc.subcore_barrier()
    pltpu.sync_copy(agg_sh.at[pl.ds(s * NPT, NPT)],
                    out_hbm.at[pl.ds(c * NPAD + s * NPT, NPT)])


@functools.lru_cache(maxsize=None)
def _sc_kernels():
    mesh = plsc.VectorSubcoreMesh(core_axis_name="c", subcore_axis_name="s")
    params = pltpu.CompilerParams(use_tc_tiling_on_sc=False)
    gather = pl.kernel(
        _sc_gather_body,
        out_type=jax.ShapeDtypeStruct((E, HID), jnp.float32),
        mesh=mesh,
        scratch_types=[
            pltpu.VMEM((NCHM, GCH), jnp.int32),
            pltpu.VMEM((TAIL,), jnp.int32),
            pltpu.VMEM((GCH, HID), jnp.float32),
            pltpu.VMEM((TAIL, HID), jnp.float32),
            pltpu.SemaphoreType.DMA,
        ],
        compiler_params=params,
    )
    scatter = pl.kernel(
        _sc_scatter_body,
        out_type=jax.ShapeDtypeStruct((SC_CORES * NPAD, HID), jnp.float32),
        mesh=mesh,
        scratch_types=[
            pltpu.VMEM((NCHM, GCH), jnp.int32),
            pltpu.VMEM((TAIL,), jnp.int32),
            pltpu.VMEM((GCH, HID), jnp.float32),
            pltpu.VMEM((TAIL, HID), jnp.float32),
            pltpu.VMEM_SHARED((NPAD, HID), jnp.float32),
            pltpu.SemaphoreType.DMA,
        ],
        compiler_params=params,
    )
    return gather, scatter


def kernel(n_feat, e_feat, edge_index, W0, b0, We1, be1, We2, be2, bc, W1,
           b1, gamma, beta_bn, Wy, by, Wy2, by2):
    srcw = edge_index[0].reshape(SC_W, EPW)
    dstw = edge_index[1].reshape(SC_W, EPW)
    srcm = srcw[:, :NCHM * GCH].reshape(SC_W, NCHM, GCH)
    srct = srcw[:, NCHM * GCH:]
    dstm = dstw[:, :NCHM * GCH].reshape(SC_W, NCHM, GCH)
    dstt = dstw[:, NCHM * GCH:]

    w0t = W0.T
    we1t = We1.T
    we2t = We2.T.astype(jnp.bfloat16)
    w1t = W1.T
    wyt = Wy.T
    wy2t = Wy2.T
    b0r = b0.reshape(1, HID)
    be1r = be1.reshape(1, E_HID)
    be2r = be2.reshape(1, HID * HID)
    b1r = b1.reshape(1, HID)
    bcr = bc.reshape(1, HID)
    gr = gamma.reshape(1, HID)
    betar = beta_bn.reshape(1, HID)
    byr = by.reshape(1, 2)
    by2r = by2.reshape(1, 2)
    rrep = jnp.repeat(jnp.eye(HID, dtype=jnp.bfloat16), HID, axis=1)
    zer = jnp.zeros((NPT, HID), jnp.float32)

    sc_gather, sc_scatter = _sc_kernels()
    out = _lin0(n_feat, w0t, b0r)
    h0 = out
    for _ in range(STEPS):
        h_src = sc_gather(out, srcm, srct)
        msg = _edge(e_feat, h_src, we1t, be1r, we2t, be2r, rrep)
        parts = sc_scatter(msg, dstm, dstt, zer).reshape(SC_CORES, NPAD, HID)
        out = _node(parts[0, :N], parts[1, :N], out, h0, w1t, b1r, bcr)
    return _bn_heads(out, gr, betar, wyt, byr, wy2t, by2r)


# speedup vs baseline: 3.1073x; 1.0629x over previous
import functools

import jax
import jax.numpy as jnp
from jax import lax
from jax.experimental import pallas as pl
from jax.experimental.pallas import tpu as pltpu
from jax.experimental.pallas import tpu_sc as plsc

N = 10000
E = 160000
D_IN = 128
HID = 32
E_IN = 16
E_HID = 128
STEPS = 2
ALPHA = 0.1
BETA = 1.0 / STEPS

SC_CORES = 2
SC_SUBCORES = 16
SC_W = SC_CORES * SC_SUBCORES
EPW = E // SC_W
GCH = 128
NCHM = 39
TAIL = EPW - NCHM * GCH
NPAD = 10240
NPT = NPAD // SC_SUBCORES

TILE_E = 2000


def _lin0_body(nf, w0t, b0, out):
    out[...] = jnp.maximum(
        jnp.dot(nf[...], w0t[...], preferred_element_type=jnp.float32) + b0[...],
        0.0)


def _edge_body(ef, h, we1t, be1, we2t, be2, rrep, msg):
    t = jnp.maximum(
        jnp.dot(ef[...], we1t[...], preferred_element_type=jnp.float32) + be1[...],
        0.0)
    wm = jnp.dot(t.astype(jnp.bfloat16), we2t[...],
                 preferred_element_type=jnp.float32) + be2[...]
    hr = jnp.dot(h[...].astype(jnp.bfloat16), rrep[...],
                 preferred_element_type=jnp.float32)
    prod = hr * wm
    t1 = prod[:, 0:128]
    for k in range(1, 8):
        t1 = t1 + prod[:, k * 128:(k + 1) * 128]
    s = t1[:, 0:64] + t1[:, 64:128]
    msg[...] = s[:, 0:32] + s[:, 32:64]


def _node_body(agg_a, agg_b, out, h0, w1t, b1, bc, new):
    conv = agg_a[...] + agg_b[...] + out[...] + bc[...]
    temp = ALPHA * conv + (1.0 - ALPHA) * h0[...]
    lin = jnp.dot(temp, w1t[...], preferred_element_type=jnp.float32) + b1[...]
    new[...] = jnp.maximum(BETA * lin + (1.0 - BETA) * temp, 0.0)


def _bn_body(x, gamma, beta_bn, wyt, by, wy2t, by2, y1, y2):
    v = x[...]
    mu = jnp.mean(v, axis=0, keepdims=True)
    d = v - mu
    var = jnp.mean(d * d, axis=0, keepdims=True)
    yb = d * (gamma[...] * lax.rsqrt(var + 1e-5)) + beta_bn[...]
    y1[...] = jax.nn.sigmoid(
        jnp.dot(yb, wyt[...], preferred_element_type=jnp.float32) + by[...])
    y2[...] = jax.nn.sigmoid(
        jnp.dot(yb, wy2t[...], preferred_element_type=jnp.float32) + by2[...])


def _lin0(n_feat, w0t, b0):
    return pl.pallas_call(
        _lin0_body,
        out_shape=jax.ShapeDtypeStruct((N, HID), jnp.float32),
    )(n_feat, w0t, b0)


def _edge(e_feat, h_src, we1t, be1, we2t, be2, rrep):
    grid = (E // TILE_E,)
    fixed = lambda i: (0, 0)
    return pl.pallas_call(
        _edge_body,
        grid=grid,
        in_specs=[
            pl.BlockSpec((TILE_E, E_IN), lambda i: (i, 0)),
            pl.BlockSpec((TILE_E, HID), lambda i: (i, 0)),
            pl.BlockSpec((E_IN, E_HID), fixed),
            pl.BlockSpec((1, E_HID), fixed),
            pl.BlockSpec((E_HID, HID * HID), fixed),
            pl.BlockSpec((1, HID * HID), fixed),
            pl.BlockSpec((HID, HID * HID), fixed),
        ],
        out_specs=pl.BlockSpec((TILE_E, HID), lambda i: (i, 0)),
        out_shape=jax.ShapeDtypeStruct((E, HID), jnp.float32),
        compiler_params=pltpu.CompilerParams(
            dimension_semantics=("arbitrary",)),
    )(e_feat, h_src, we1t, be1, we2t, be2, rrep)


def _node(agg_a, agg_b, out, h0, w1t, b1, bc):
    return pl.pallas_call(
        _node_body,
        out_shape=jax.ShapeDtypeStruct((N, HID), jnp.float32),
    )(agg_a, agg_b, out, h0, w1t, b1, bc)


def _bn_heads(x, gamma, beta_bn, wyt, by, wy2t, by2):
    return pl.pallas_call(
        _bn_body,
        out_shape=(jax.ShapeDtypeStruct((N, 2), jnp.float32),
                   jax.ShapeDtypeStruct((N, 2), jnp.float32)),
    )(x, gamma, beta_bn, wyt, by, wy2t, by2)


def _sc_gather_body(table_hbm, idxm_hbm, idxt_hbm, out_hbm, idx_v, idxt_v,
                    rows_a, rows_b, rowst_v, gsem_a, gsem_b, ssem_a, ssem_b,
                    tsem):
    c = lax.axis_index("c")
    s = lax.axis_index("s")
    wid = s * SC_CORES + c
    base = wid * EPW
    pltpu.sync_copy(idxm_hbm.at[wid], idx_v)
    pltpu.sync_copy(idxt_hbm.at[wid], idxt_v)

    def issue(j, buf, sem):
        pltpu.async_copy(table_hbm.at[idx_v.at[j]], buf, sem)

    def wait_g(buf, sem):
        pltpu.make_async_copy(table_hbm.at[idx_v.at[0]], buf, sem).wait()

    def store(j, buf, sem):
        pltpu.async_copy(buf, out_hbm.at[pl.ds(base + j * GCH, GCH)], sem)

    def wait_s(buf, sem):
        pltpu.make_async_copy(out_hbm.at[pl.ds(base, GCH)], buf, sem).wait()

    issue(0, rows_a, gsem_a)
    issue(1, rows_b, gsem_b)

    def body(g, carry):
        j0 = 2 * g
        wait_g(rows_a, gsem_a)
        store(j0, rows_a, ssem_a)
        wait_g(rows_b, gsem_b)
        store(j0 + 1, rows_b, ssem_b)

        @pl.when(j0 + 2 < NCHM)
        def _():
            wait_s(rows_a, ssem_a)
            issue(j0 + 2, rows_a, gsem_a)

        @pl.when(j0 + 3 < NCHM)
        def _():
            wait_s(rows_b, ssem_b)
            issue(j0 + 3, rows_b, gsem_b)

        return carry

    lax.fori_loop(0, (NCHM - 1) // 2, body, 0)
    wait_g(rows_a, gsem_a)
    store(NCHM - 1, rows_a, ssem_a)
    pltpu.async_copy(table_hbm.at[idxt_v], rowst_v, tsem).wait()
    pltpu.sync_copy(rowst_v, out_hbm.at[pl.ds(base + NCHM * GCH, TAIL)])
    wait_s(rows_a, ssem_a)
    wait_s(rows_b, ssem_b)


def _sc_scatter_body(msg_hbm, idxm_hbm, idxt_hbm, zer_hbm, out_hbm, idx_v,
                     idxt_v, rows_a, rows_b, rowst_v, agg_sh, lsem_a, lsem_b,
                     tsem):
    c = lax.axis_index("c")
    s = lax.axis_index("s")
    wid = s * SC_CORES + c
    base = wid * EPW
    pltpu.sync_copy(zer_hbm, agg_sh.at[pl.ds(s * NPT, NPT)])
    pltpu.sync_copy(idxm_hbm.at[wid], idx_v)
    pltpu.sync_copy(idxt_hbm.at[wid], idxt_v)
    plsc.subcore_barrier()

    def load(j, buf, sem):
        pltpu.async_copy(msg_hbm.at[pl.ds(base + j * GCH, GCH)], buf, sem)

    def wait_l(buf, sem):
        pltpu.make_async_copy(msg_hbm.at[pl.ds(base, GCH)], buf, sem).wait()

    load(0, rows_a, lsem_a)
    load(1, rows_b, lsem_b)

    def body(g, carry):
        j0 = 2 * g
        wait_l(rows_a, lsem_a)
        pltpu.sync_copy(rows_a, agg_sh.at[idx_v.at[j0]], add=True)

        @pl.when(j0 + 2 < NCHM)
        def _():
            load(j0 + 2, rows_a, lsem_a)

        wait_l(rows_b, lsem_b)
        pltpu.sync_copy(rows_b, agg_sh.at[idx_v.at[j0 + 1]], add=True)

        @pl.when(j0 + 3 < NCHM)
        def _():
            load(j0 + 3, rows_b, lsem_b)

        return carry

    lax.fori_loop(0, (NCHM - 1) // 2, body, 0)
    wait_l(rows_a, lsem_a)
    pltpu.sync_copy(rows_a, agg_sh.at[idx_v.at[NCHM - 1]], add=True)
    pltpu.async_copy(
        msg_hbm.at[pl.ds(base + NCHM * GCH, TAIL)], rowst_v, tsem).wait()
    pltpu.sync_copy(rowst_v, agg_sh.at[idxt_v], add=True)
    plsc.subcore_barrier()
    pltpu.sync_copy(agg_sh.at[pl.ds(s * NPT, NPT)],
                    out_hbm.at[pl.ds(c * NPAD + s * NPT, NPT)])


@functools.lru_cache(maxsize=None)
def _sc_kernels():
    mesh = plsc.VectorSubcoreMesh(core_axis_name="c", subcore_axis_name="s")
    params = pltpu.CompilerParams(use_tc_tiling_on_sc=False)
    gather = pl.kernel(
        _sc_gather_body,
        out_type=jax.ShapeDtypeStruct((E, HID), jnp.float32),
        mesh=mesh,
        scratch_types=[
            pltpu.VMEM((NCHM, GCH), jnp.int32),
            pltpu.VMEM((TAIL,), jnp.int32),
            pltpu.VMEM((GCH, HID), jnp.float32),
            pltpu.VMEM((GCH, HID), jnp.float32),
            pltpu.VMEM((TAIL, HID), jnp.float32),
            pltpu.SemaphoreType.DMA,
            pltpu.SemaphoreType.DMA,
            pltpu.SemaphoreType.DMA,
            pltpu.SemaphoreType.DMA,
            pltpu.SemaphoreType.DMA,
        ],
        compiler_params=params,
    )
    scatter = pl.kernel(
        _sc_scatter_body,
        out_type=jax.ShapeDtypeStruct((SC_CORES * NPAD, HID), jnp.float32),
        mesh=mesh,
        scratch_types=[
            pltpu.VMEM((NCHM, GCH), jnp.int32),
            pltpu.VMEM((TAIL,), jnp.int32),
            pltpu.VMEM((GCH, HID), jnp.float32),
            pltpu.VMEM((GCH, HID), jnp.float32),
            pltpu.VMEM((TAIL, HID), jnp.float32),
            pltpu.VMEM_SHARED((NPAD, HID), jnp.float32),
            pltpu.SemaphoreType.DMA,
            pltpu.SemaphoreType.DMA,
            pltpu.SemaphoreType.DMA,
        ],
        compiler_params=params,
    )
    return gather, scatter


def kernel(n_feat, e_feat, edge_index, W0, b0, We1, be1, We2, be2, bc, W1,
           b1, gamma, beta_bn, Wy, by, Wy2, by2):
    srcw = edge_index[0].reshape(SC_W, EPW)
    dstw = edge_index[1].reshape(SC_W, EPW)
    srcm = srcw[:, :NCHM * GCH].reshape(SC_W, NCHM, GCH)
    srct = srcw[:, NCHM * GCH:]
    dstm = dstw[:, :NCHM * GCH].reshape(SC_W, NCHM, GCH)
    dstt = dstw[:, NCHM * GCH:]

    w0t = W0.T
    we1t = We1.T
    we2t = We2.T.astype(jnp.bfloat16)
    w1t = W1.T
    wyt = Wy.T
    wy2t = Wy2.T
    b0r = b0.reshape(1, HID)
    be1r = be1.reshape(1, E_HID)
    be2r = be2.reshape(1, HID * HID)
    b1r = b1.reshape(1, HID)
    bcr = bc.reshape(1, HID)
    gr = gamma.reshape(1, HID)
    betar = beta_bn.reshape(1, HID)
    byr = by.reshape(1, 2)
    by2r = by2.reshape(1, 2)
    rrep = jnp.repeat(jnp.eye(HID, dtype=jnp.bfloat16), HID, axis=1)
    zer = jnp.zeros((NPT, HID), jnp.float32)

    sc_gather, sc_scatter = _sc_kernels()
    out = _lin0(n_feat, w0t, b0r)
    h0 = out
    for _ in range(STEPS):
        h_src = sc_gather(out, srcm, srct)
        msg = _edge(e_feat, h_src, we1t, be1r, we2t, be2r, rrep)
        parts = sc_scatter(msg, dstm, dstt, zer).reshape(SC_CORES, NPAD, HID)
        out = _node(parts[0, :N], parts[1, :N], out, h0, w1t, b1r, bcr)
    return _bn_heads(out, gr, betar, wyt, byr, wy2t, by2r)


# speedup vs baseline: 3.4896x; 1.1230x over previous
import functools

import jax
import jax.numpy as jnp
from jax import lax
from jax.experimental import pallas as pl
from jax.experimental.pallas import tpu as pltpu
from jax.experimental.pallas import tpu_sc as plsc

N = 10000
E = 160000
D_IN = 128
HID = 32
E_IN = 16
E_HID = 128
STEPS = 2
ALPHA = 0.1
BETA = 1.0 / STEPS

SC_CORES = 2
SC_SUBCORES = 16
SC_W = SC_CORES * SC_SUBCORES
EPW = E // SC_W
GCH = 128
NCHM = 39
TAIL = EPW - NCHM * GCH
NPAD = 10240
NPT = NPAD // SC_SUBCORES

TILE_E = 1600


def _lin0_body(nf, w0t, b0, out):
    out[...] = jnp.maximum(
        jnp.dot(nf[...], w0t[...], preferred_element_type=jnp.float32) + b0[...],
        0.0)


def _edge_body(ef, hp, we1t, be1, we2t, b2, rrep, msgp):
    hpv = hp[...]
    h = jnp.concatenate([hpv[:, HID * q:HID * (q + 1)] for q in range(4)],
                        axis=0)
    t = jnp.maximum(
        jnp.dot(ef[...], we1t[...], preferred_element_type=jnp.float32) + be1[...],
        0.0)
    wm = jnp.dot(t.astype(jnp.bfloat16), we2t[...],
                 preferred_element_type=jnp.float32)
    hr = jnp.dot(h.astype(jnp.bfloat16), rrep[...],
                 preferred_element_type=jnp.float32)
    prod = hr * wm
    t1 = prod[:, 0:128]
    for k in range(1, 8):
        t1 = t1 + prod[:, k * 128:(k + 1) * 128]
    s = t1[:, 0:64] + t1[:, 64:128]
    m2 = jnp.dot(h, b2[...], preferred_element_type=jnp.float32)
    msg = s[:, 0:32] + s[:, 32:64] + m2
    q4 = TILE_E // 4
    msgp[...] = jnp.concatenate([msg[q4 * q:q4 * (q + 1), :] for q in range(4)],
                                axis=1)


def _node_body(agg_a, agg_b, out, h0, w1t, b1, bc, new):
    conv = agg_a[...] + agg_b[...] + out[...] + bc[...]
    temp = ALPHA * conv + (1.0 - ALPHA) * h0[...]
    lin = jnp.dot(temp, w1t[...], preferred_element_type=jnp.float32) + b1[...]
    new[...] = jnp.maximum(BETA * lin + (1.0 - BETA) * temp, 0.0)


def _bn_body(x, gamma, beta_bn, wyt, by, wy2t, by2, y1, y2):
    v = x[...]
    mu = jnp.mean(v, axis=0, keepdims=True)
    d = v - mu
    var = jnp.mean(d * d, axis=0, keepdims=True)
    yb = d * (gamma[...] * lax.rsqrt(var + 1e-5)) + beta_bn[...]
    y1[...] = jax.nn.sigmoid(
        jnp.dot(yb, wyt[...], preferred_element_type=jnp.float32) + by[...])
    y2[...] = jax.nn.sigmoid(
        jnp.dot(yb, wy2t[...], preferred_element_type=jnp.float32) + by2[...])


def _lin0(n_feat, w0t, b0):
    return pl.pallas_call(
        _lin0_body,
        out_shape=jax.ShapeDtypeStruct((N, HID), jnp.float32),
    )(n_feat, w0t, b0)


def _edge(e_feat, hp, we1t, be1, we2t, b2, rrep):
    grid = (E // TILE_E,)
    fixed = lambda i: (0, 0)
    return pl.pallas_call(
        _edge_body,
        grid=grid,
        in_specs=[
            pl.BlockSpec((TILE_E, E_IN), lambda i: (i, 0)),
            pl.BlockSpec((TILE_E // 4, 4 * HID), lambda i: (i, 0)),
            pl.BlockSpec((E_IN, E_HID), fixed),
            pl.BlockSpec((1, E_HID), fixed),
            pl.BlockSpec((E_HID, HID * HID), fixed),
            pl.BlockSpec((HID, HID), fixed),
            pl.BlockSpec((HID, HID * HID), fixed),
        ],
        out_specs=pl.BlockSpec((TILE_E // 4, 4 * HID), lambda i: (i, 0)),
        out_shape=jax.ShapeDtypeStruct((E // 4, 4 * HID), jnp.float32),
        compiler_params=pltpu.CompilerParams(
            dimension_semantics=("arbitrary",)),
    )(e_feat, hp, we1t, be1, we2t, b2, rrep)


def _node(agg_a, agg_b, out, h0, w1t, b1, bc):
    return pl.pallas_call(
        _node_body,
        out_shape=jax.ShapeDtypeStruct((N, HID), jnp.float32),
    )(agg_a, agg_b, out, h0, w1t, b1, bc)


def _bn_heads(x, gamma, beta_bn, wyt, by, wy2t, by2):
    return pl.pallas_call(
        _bn_body,
        out_shape=(jax.ShapeDtypeStruct((N, 2), jnp.float32),
                   jax.ShapeDtypeStruct((N, 2), jnp.float32)),
    )(x, gamma, beta_bn, wyt, by, wy2t, by2)


def _sc_gather_body(table_hbm, idxm_hbm, idxt_hbm, out_hbm, idx_v, idxt_v,
                    rows_a, rows_b, rowst_v, gsem_a, gsem_b, ssem_a, ssem_b,
                    tsem):
    c = lax.axis_index("c")
    s = lax.axis_index("s")
    wid = s * SC_CORES + c
    base = wid * EPW
    pltpu.sync_copy(idxm_hbm.at[wid], idx_v)
    pltpu.sync_copy(idxt_hbm.at[wid], idxt_v)

    def issue(j, buf, sem):
        pltpu.async_copy(table_hbm.at[idx_v.at[j]], buf, sem)

    def wait_g(buf, sem):
        pltpu.make_async_copy(table_hbm.at[idx_v.at[0]], buf, sem).wait()

    def store(j, buf, sem):
        pltpu.async_copy(buf, out_hbm.at[pl.ds(base + j * GCH, GCH)], sem)

    def wait_s(buf, sem):
        pltpu.make_async_copy(out_hbm.at[pl.ds(base, GCH)], buf, sem).wait()

    issue(0, rows_a, gsem_a)
    issue(1, rows_b, gsem_b)

    def body(g, carry):
        j0 = 2 * g
        wait_g(rows_a, gsem_a)
        store(j0, rows_a, ssem_a)
        wait_g(rows_b, gsem_b)
        store(j0 + 1, rows_b, ssem_b)

        @pl.when(j0 + 2 < NCHM)
        def _():
            wait_s(rows_a, ssem_a)
            issue(j0 + 2, rows_a, gsem_a)

        @pl.when(j0 + 3 < NCHM)
        def _():
            wait_s(rows_b, ssem_b)
            issue(j0 + 3, rows_b, gsem_b)

        return carry

    lax.fori_loop(0, (NCHM - 1) // 2, body, 0)
    wait_g(rows_a, gsem_a)
    store(NCHM - 1, rows_a, ssem_a)
    pltpu.async_copy(table_hbm.at[idxt_v], rowst_v, tsem).wait()
    pltpu.sync_copy(rowst_v, out_hbm.at[pl.ds(base + NCHM * GCH, TAIL)])
    wait_s(rows_a, ssem_a)
    wait_s(rows_b, ssem_b)


def _sc_scatter_body(msg_hbm, idxm_hbm, idxt_hbm, zer_hbm, out_hbm, idx_v,
                     idxt_v, rows_a, rows_b, rowst_v, agg_sh, lsem_a, lsem_b,
                     tsem):
    c = lax.axis_index("c")
    s = lax.axis_index("s")
    wid = s * SC_CORES + c
    base = wid * EPW
    pltpu.sync_copy(zer_hbm, agg_sh.at[pl.ds(s * NPT, NPT)])
    pltpu.sync_copy(idxm_hbm.at[wid], idx_v)
    pltpu.sync_copy(idxt_hbm.at[wid], idxt_v)
    plsc.subcore_barrier()

    def load(j, buf, sem):
        pltpu.async_copy(msg_hbm.at[pl.ds(base + j * GCH, GCH)], buf, sem)

    def wait_l(buf, sem):
        pltpu.make_async_copy(msg_hbm.at[pl.ds(base, GCH)], buf, sem).wait()

    load(0, rows_a, lsem_a)
    load(1, rows_b, lsem_b)

    def body(g, carry):
        j0 = 2 * g
        wait_l(rows_a, lsem_a)
        pltpu.sync_copy(rows_a, agg_sh.at[idx_v.at[j0]], add=True)

        @pl.when(j0 + 2 < NCHM)
        def _():
            load(j0 + 2, rows_a, lsem_a)

        wait_l(rows_b, lsem_b)
        pltpu.sync_copy(rows_b, agg_sh.at[idx_v.at[j0 + 1]], add=True)

        @pl.when(j0 + 3 < NCHM)
        def _():
            load(j0 + 3, rows_b, lsem_b)

        return carry

    lax.fori_loop(0, (NCHM - 1) // 2, body, 0)
    wait_l(rows_a, lsem_a)
    pltpu.sync_copy(rows_a, agg_sh.at[idx_v.at[NCHM - 1]], add=True)
    pltpu.async_copy(
        msg_hbm.at[pl.ds(base + NCHM * GCH, TAIL)], rowst_v, tsem).wait()
    pltpu.sync_copy(rowst_v, agg_sh.at[idxt_v], add=True)
    plsc.subcore_barrier()
    pltpu.sync_copy(agg_sh.at[pl.ds(s * NPT, NPT)],
                    out_hbm.at[pl.ds(c * NPAD + s * NPT, NPT)])


@functools.lru_cache(maxsize=None)
def _sc_kernels():
    mesh = plsc.VectorSubcoreMesh(core_axis_name="c", subcore_axis_name="s")
    params = pltpu.CompilerParams(use_tc_tiling_on_sc=False)
    gather = pl.kernel(
        _sc_gather_body,
        out_type=jax.ShapeDtypeStruct((E, HID), jnp.float32),
        mesh=mesh,
        scratch_types=[
            pltpu.VMEM((NCHM, GCH), jnp.int32),
            pltpu.VMEM((TAIL,), jnp.int32),
            pltpu.VMEM((GCH, HID), jnp.float32),
            pltpu.VMEM((GCH, HID), jnp.float32),
            pltpu.VMEM((TAIL, HID), jnp.float32),
            pltpu.SemaphoreType.DMA,
            pltpu.SemaphoreType.DMA,
            pltpu.SemaphoreType.DMA,
            pltpu.SemaphoreType.DMA,
            pltpu.SemaphoreType.DMA,
        ],
        compiler_params=params,
    )
    scatter = pl.kernel(
        _sc_scatter_body,
        out_type=jax.ShapeDtypeStruct((SC_CORES * NPAD, HID), jnp.float32),
        mesh=mesh,
        scratch_types=[
            pltpu.VMEM((NCHM, GCH), jnp.int32),
            pltpu.VMEM((TAIL,), jnp.int32),
            pltpu.VMEM((GCH, HID), jnp.float32),
            pltpu.VMEM((GCH, HID), jnp.float32),
            pltpu.VMEM((TAIL, HID), jnp.float32),
            pltpu.VMEM_SHARED((NPAD, HID), jnp.float32),
            pltpu.SemaphoreType.DMA,
            pltpu.SemaphoreType.DMA,
            pltpu.SemaphoreType.DMA,
        ],
        compiler_params=params,
    )
    return gather, scatter


def kernel(n_feat, e_feat, edge_index, W0, b0, We1, be1, We2, be2, bc, W1,
           b1, gamma, beta_bn, Wy, by, Wy2, by2):
    def _to_slots(x):
        return x.reshape(E // TILE_E, 4, TILE_E // 4).transpose(0, 2, 1)

    srcw = _to_slots(edge_index[0]).reshape(SC_W, EPW)
    dstw = _to_slots(edge_index[1]).reshape(SC_W, EPW)
    srcm = srcw[:, :NCHM * GCH].reshape(SC_W, NCHM, GCH)
    srct = srcw[:, NCHM * GCH:]
    dstm = dstw[:, :NCHM * GCH].reshape(SC_W, NCHM, GCH)
    dstt = dstw[:, NCHM * GCH:]

    w0t = W0.T
    we1t = We1.T
    we2t = We2.T.astype(jnp.bfloat16)
    w1t = W1.T
    wyt = Wy.T
    wy2t = Wy2.T
    b0r = b0.reshape(1, HID)
    be1r = be1.reshape(1, E_HID)
    b2 = be2.reshape(HID, HID)
    b1r = b1.reshape(1, HID)
    bcr = bc.reshape(1, HID)
    gr = gamma.reshape(1, HID)
    betar = beta_bn.reshape(1, HID)
    byr = by.reshape(1, 2)
    by2r = by2.reshape(1, 2)
    rrep = jnp.repeat(jnp.eye(HID, dtype=jnp.bfloat16), HID, axis=1)
    zer = jnp.zeros((NPT, HID), jnp.float32)

    sc_gather, sc_scatter = _sc_kernels()
    out = _lin0(n_feat, w0t, b0r)
    h0 = out
    for _ in range(STEPS):
        hp = sc_gather(out, srcm, srct).reshape(E // 4, 4 * HID)
        msgp = _edge(e_feat, hp, we1t, be1r, we2t, b2, rrep)
        parts = sc_scatter(msgp.reshape(E, HID), dstm, dstt,
                           zer).reshape(SC_CORES, NPAD, HID)
        out = _node(parts[0, :N], parts[1, :N], out, h0, w1t, b1r, bcr)
    return _bn_heads(out, gr, betar, wyt, byr, wy2t, by2r)


# speedup vs baseline: 3.6368x; 1.0422x over previous
import functools

import jax
import jax.numpy as jnp
from jax import lax
from jax.experimental import pallas as pl
from jax.experimental.pallas import tpu as pltpu
from jax.experimental.pallas import tpu_sc as plsc

N = 10000
E = 160000
D_IN = 128
HID = 32
E_IN = 16
E_HID = 128
STEPS = 2
ALPHA = 0.1
BETA = 1.0 / STEPS

SC_CORES = 2
SC_SUBCORES = 16
SC_W = SC_CORES * SC_SUBCORES
EPW = E // SC_W
GCH = 128
NCHM = 39
TAIL = EPW - NCHM * GCH
NPAD = 10240
NPT = NPAD // SC_SUBCORES

TILE_E = 3200


def _lin0_body(nf, w0t, b0, out):
    out[...] = jnp.maximum(
        jnp.dot(nf[...], w0t[...], preferred_element_type=jnp.float32) + b0[...],
        0.0)


def _edge_body(ef, hp, we1t, be1, we2t, b2, rrep, stile, msgp):
    hpv = hp[...]
    h = jnp.concatenate([hpv[:, HID * q:HID * (q + 1)] for q in range(4)],
                        axis=0)
    t = jnp.maximum(
        jnp.dot(ef[...], we1t[...], preferred_element_type=jnp.float32) + be1[...],
        0.0)
    wm = jnp.dot(t.astype(jnp.bfloat16), we2t[...],
                 preferred_element_type=jnp.float32)
    hr = jnp.dot(h.astype(jnp.bfloat16), rrep[...],
                 preferred_element_type=jnp.float32)
    prod = hr * wm
    t1 = prod[:, 0:128]
    for k in range(1, 8):
        t1 = t1 + prod[:, k * 128:(k + 1) * 128]
    s = t1[:, 0:64] + t1[:, 64:128]
    m2 = jnp.dot(h, b2[...], preferred_element_type=jnp.float32)
    msg = s[:, 0:32] + s[:, 32:64] + m2
    del stile
    q4 = TILE_E // 4
    msgp[...] = jnp.concatenate([msg[q4 * q:q4 * (q + 1), :] for q in range(4)],
                                axis=1)


def _node_body(agg_a, agg_b, out, h0, w1t, b1, bc, new):
    conv = agg_a[...] + agg_b[...] + out[...] + bc[...]
    temp = ALPHA * conv + (1.0 - ALPHA) * h0[...]
    lin = jnp.dot(temp, w1t[...], preferred_element_type=jnp.float32) + b1[...]
    new[...] = jnp.maximum(BETA * lin + (1.0 - BETA) * temp, 0.0)


def _bn_body(x, gamma, beta_bn, wyt, by, wy2t, by2, y1, y2):
    v = x[...]
    mu = jnp.mean(v, axis=0, keepdims=True)
    d = v - mu
    var = jnp.mean(d * d, axis=0, keepdims=True)
    yb = d * (gamma[...] * lax.rsqrt(var + 1e-5)) + beta_bn[...]
    y1[...] = jax.nn.sigmoid(
        jnp.dot(yb, wyt[...], preferred_element_type=jnp.float32) + by[...])
    y2[...] = jax.nn.sigmoid(
        jnp.dot(yb, wy2t[...], preferred_element_type=jnp.float32) + by2[...])


def _lin0(n_feat, w0t, b0):
    return pl.pallas_call(
        _lin0_body,
        out_shape=jax.ShapeDtypeStruct((N, HID), jnp.float32),
    )(n_feat, w0t, b0)


def _edge(e_feat, hp, we1t, be1, we2t, b2, rrep, stile):
    grid = (E // TILE_E,)
    fixed = lambda i: (0, 0)
    return pl.pallas_call(
        _edge_body,
        grid=grid,
        in_specs=[
            pl.BlockSpec((TILE_E, E_IN), lambda i: (i, 0)),
            pl.BlockSpec((TILE_E // 4, 4 * HID), lambda i: (i, 0)),
            pl.BlockSpec((E_IN, E_HID), fixed),
            pl.BlockSpec((1, E_HID), fixed),
            pl.BlockSpec((E_HID, HID * HID), fixed),
            pl.BlockSpec((HID, HID), fixed),
            pl.BlockSpec((HID, HID * HID), fixed),
            pl.BlockSpec((HID * HID, HID), fixed),
        ],
        out_specs=pl.BlockSpec((TILE_E // 4, 4 * HID), lambda i: (i, 0)),
        out_shape=jax.ShapeDtypeStruct((E // 4, 4 * HID), jnp.float32),
        compiler_params=pltpu.CompilerParams(
            dimension_semantics=("arbitrary",)),
    )(e_feat, hp, we1t, be1, we2t, b2, rrep, stile)


def _node(agg_a, agg_b, out, h0, w1t, b1, bc):
    return pl.pallas_call(
        _node_body,
        out_shape=jax.ShapeDtypeStruct((N, HID), jnp.float32),
    )(agg_a, agg_b, out, h0, w1t, b1, bc)


def _bn_heads(x, gamma, beta_bn, wyt, by, wy2t, by2):
    return pl.pallas_call(
        _bn_body,
        out_shape=(jax.ShapeDtypeStruct((N, 2), jnp.float32),
                   jax.ShapeDtypeStruct((N, 2), jnp.float32)),
    )(x, gamma, beta_bn, wyt, by, wy2t, by2)


def _sc_gather_body(table_hbm, idxm_hbm, idxt_hbm, out_hbm, idx_v, idxt_v,
                    rows_a, rows_b, rowst_v, gsem_a, gsem_b, ssem_a, ssem_b,
                    tsem):
    c = lax.axis_index("c")
    s = lax.axis_index("s")
    wid = s * SC_CORES + c
    base = wid * EPW
    pltpu.sync_copy(idxm_hbm.at[wid], idx_v)
    pltpu.sync_copy(idxt_hbm.at[wid], idxt_v)

    def issue(j, buf, sem):
        pltpu.async_copy(table_hbm.at[idx_v.at[j]], buf, sem)

    def wait_g(buf, sem):
        pltpu.make_async_copy(table_hbm.at[idx_v.at[0]], buf, sem).wait()

    def store(j, buf, sem):
        pltpu.async_copy(buf, out_hbm.at[pl.ds(base + j * GCH, GCH)], sem)

    def wait_s(buf, sem):
        pltpu.make_async_copy(out_hbm.at[pl.ds(base, GCH)], buf, sem).wait()

    issue(0, rows_a, gsem_a)
    issue(1, rows_b, gsem_b)

    def body(g, carry):
        j0 = 2 * g
        wait_g(rows_a, gsem_a)
        store(j0, rows_a, ssem_a)
        wait_g(rows_b, gsem_b)
        store(j0 + 1, rows_b, ssem_b)

        @pl.when(j0 + 2 < NCHM)
        def _():
            wait_s(rows_a, ssem_a)
            issue(j0 + 2, rows_a, gsem_a)

        @pl.when(j0 + 3 < NCHM)
        def _():
            wait_s(rows_b, ssem_b)
            issue(j0 + 3, rows_b, gsem_b)

        return carry

    lax.fori_loop(0, (NCHM - 1) // 2, body, 0)
    wait_g(rows_a, gsem_a)
    store(NCHM - 1, rows_a, ssem_a)
    pltpu.async_copy(table_hbm.at[idxt_v], rowst_v, tsem).wait()
    pltpu.sync_copy(rowst_v, out_hbm.at[pl.ds(base + NCHM * GCH, TAIL)])
    wait_s(rows_a, ssem_a)
    wait_s(rows_b, ssem_b)


def _sc_scatter_body(msg_hbm, idxm_hbm, idxt_hbm, zer_hbm, out_hbm, idx_v,
                     idxt_v, rows_a, rows_b, rowst_v, agg_sh, lsem_a, lsem_b,
                     tsem):
    c = lax.axis_index("c")
    s = lax.axis_index("s")
    wid = s * SC_CORES + c
    base = wid * EPW
    pltpu.sync_copy(zer_hbm, agg_sh.at[pl.ds(s * NPT, NPT)])
    pltpu.sync_copy(idxm_hbm.at[wid], idx_v)
    pltpu.sync_copy(idxt_hbm.at[wid], idxt_v)
    plsc.subcore_barrier()

    def load(j, buf, sem):
        pltpu.async_copy(msg_hbm.at[pl.ds(base + j * GCH, GCH)], buf, sem)

    def wait_l(buf, sem):
        pltpu.make_async_copy(msg_hbm.at[pl.ds(base, GCH)], buf, sem).wait()

    load(0, rows_a, lsem_a)
    load(1, rows_b, lsem_b)

    def body(g, carry):
        j0 = 2 * g
        wait_l(rows_a, lsem_a)
        pltpu.sync_copy(rows_a, agg_sh.at[idx_v.at[j0]], add=True)

        @pl.when(j0 + 2 < NCHM)
        def _():
            load(j0 + 2, rows_a, lsem_a)

        wait_l(rows_b, lsem_b)
        pltpu.sync_copy(rows_b, agg_sh.at[idx_v.at[j0 + 1]], add=True)

        @pl.when(j0 + 3 < NCHM)
        def _():
            load(j0 + 3, rows_b, lsem_b)

        return carry

    lax.fori_loop(0, (NCHM - 1) // 2, body, 0)
    wait_l(rows_a, lsem_a)
    pltpu.sync_copy(rows_a, agg_sh.at[idx_v.at[NCHM - 1]], add=True)
    pltpu.async_copy(
        msg_hbm.at[pl.ds(base + NCHM * GCH, TAIL)], rowst_v, tsem).wait()
    pltpu.sync_copy(rowst_v, agg_sh.at[idxt_v], add=True)
    plsc.subcore_barrier()
    pltpu.sync_copy(agg_sh.at[pl.ds(s * NPT, NPT)],
                    out_hbm.at[pl.ds(c * NPAD + s * NPT, NPT)])


@functools.lru_cache(maxsize=None)
def _sc_kernels():
    mesh = plsc.VectorSubcoreMesh(core_axis_name="c", subcore_axis_name="s")
    params = pltpu.CompilerParams(use_tc_tiling_on_sc=False)
    gather = pl.kernel(
        _sc_gather_body,
        out_type=jax.ShapeDtypeStruct((E, HID), jnp.float32),
        mesh=mesh,
        scratch_types=[
            pltpu.VMEM((NCHM, GCH), jnp.int32),
            pltpu.VMEM((TAIL,), jnp.int32),
            pltpu.VMEM((GCH, HID), jnp.float32),
            pltpu.VMEM((GCH, HID), jnp.float32),
            pltpu.VMEM((TAIL, HID), jnp.float32),
            pltpu.SemaphoreType.DMA,
            pltpu.SemaphoreType.DMA,
            pltpu.SemaphoreType.DMA,
            pltpu.SemaphoreType.DMA,
            pltpu.SemaphoreType.DMA,
        ],
        compiler_params=params,
    )
    scatter = pl.kernel(
        _sc_scatter_body,
        out_type=jax.ShapeDtypeStruct((SC_CORES * NPAD, HID), jnp.float32),
        mesh=mesh,
        scratch_types=[
            pltpu.VMEM((NCHM, GCH), jnp.int32),
            pltpu.VMEM((TAIL,), jnp.int32),
            pltpu.VMEM((GCH, HID), jnp.float32),
            pltpu.VMEM((GCH, HID), jnp.float32),
            pltpu.VMEM((TAIL, HID), jnp.float32),
            pltpu.VMEM_SHARED((NPAD, HID), jnp.float32),
            pltpu.SemaphoreType.DMA,
            pltpu.SemaphoreType.DMA,
            pltpu.SemaphoreType.DMA,
        ],
        compiler_params=params,
    )
    return gather, scatter


def kernel(n_feat, e_feat, edge_index, W0, b0, We1, be1, We2, be2, bc, W1,
           b1, gamma, beta_bn, Wy, by, Wy2, by2):
    def _to_slots(x):
        return x.reshape(E // TILE_E, 4, TILE_E // 4).transpose(0, 2, 1)

    srcw = _to_slots(edge_index[0]).reshape(SC_W, EPW)
    dstw = _to_slots(edge_index[1]).reshape(SC_W, EPW)
    srcm = srcw[:, :NCHM * GCH].reshape(SC_W, NCHM, GCH)
    srct = srcw[:, NCHM * GCH:]
    dstm = dstw[:, :NCHM * GCH].reshape(SC_W, NCHM, GCH)
    dstt = dstw[:, NCHM * GCH:]

    w0t = W0.T
    we1t = We1.T
    we2t = We2.T.astype(jnp.bfloat16)
    w1t = W1.T
    wyt = Wy.T
    wy2t = Wy2.T
    b0r = b0.reshape(1, HID)
    be1r = be1.reshape(1, E_HID)
    b2 = be2.reshape(HID, HID)
    b1r = b1.reshape(1, HID)
    bcr = bc.reshape(1, HID)
    gr = gamma.reshape(1, HID)
    betar = beta_bn.reshape(1, HID)
    byr = by.reshape(1, 2)
    by2r = by2.reshape(1, 2)
    rrep = jnp.repeat(jnp.eye(HID, dtype=jnp.bfloat16), HID, axis=1)
    stile = jnp.tile(jnp.eye(HID, dtype=jnp.bfloat16), (HID, 1))
    zer = jnp.zeros((NPT, HID), jnp.float32)

    sc_gather, sc_scatter = _sc_kernels()
    out = _lin0(n_feat, w0t, b0r)
    h0 = out
    for _ in range(STEPS):
        hp = sc_gather(out, srcm, srct).reshape(E // 4, 4 * HID)
        msgp = _edge(e_feat, hp, we1t, be1r, we2t, b2, rrep, stile)
        parts = sc_scatter(msgp.reshape(E, HID), dstm, dstt,
                           zer).reshape(SC_CORES, NPAD, HID)
        out = _node(parts[0, :N], parts[1, :N], out, h0, w1t, b1r, bcr)
    return _bn_heads(out, gr, betar, wyt, byr, wy2t, by2r)
